# baseline (speedup 1.0000x reference)
import functools

import jax
import jax.numpy as jnp
from jax import lax
from jax.experimental import pallas as pl
from jax.experimental.pallas import tpu as pltpu
from jax.experimental.pallas import tpu_sc as plsc

N = 10000
E = 320000
D = 128
H = 128
C = 64

NC = 2
NS = 16
NW = NC * NS
CHUNK = 80
NCHUNK = 125
SB = 25
NSB = NCHUNK // SB
NA = 10240
RPS = NA // NS
NPADR = NA - N

_MESH = plsc.VectorSubcoreMesh(core_axis_name="c", subcore_axis_name="s")


def _sc_agg_body(with_deg, *refs):
    if with_deg:
        (y_hbm, e_hbm, z_hbm, z1_hbm, part_hbm, deg_hbm,
         srcv2, dstv2, rows0, rows1, ones1, acc, dacc,
         sg0, sg1, ss0, ss1) = refs
    else:
        (y_hbm, e_hbm, z_hbm, part_hbm,
         srcv2, dstv2, rows0, rows1, acc,
         sg0, sg1, ss0, ss1) = refs
    cid = lax.axis_index("c")
    sid = lax.axis_index("s")
    wid = cid * NS + sid
    r0 = sid * RPS

    pltpu.sync_copy(z_hbm, acc.at[pl.ds(r0, RPS)])
    if with_deg:
        pltpu.sync_copy(z1_hbm, dacc.at[pl.ds(r0, RPS)])

        @pl.loop(0, CHUNK // 16)
        def _(i):
            ones1[pl.ds(i * 16, 16)] = jnp.full((16,), 1.0, jnp.float32)

    plsc.subcore_barrier()

    def gather(c, rows, sem):
        return pltpu.async_copy(y_hbm.at[srcv2.at[c]], rows, sem)

    def wait_gather(c, rows, sem):
        pltpu.make_async_copy(y_hbm.at[srcv2.at[c]], rows, sem).wait()

    def scatter(c, rows, sem):
        return pltpu.async_copy(rows, acc.at[dstv2.at[c]], sem, add=True)

    def wait_scatter(c, rows, sem):
        pltpu.make_async_copy(rows, acc.at[dstv2.at[c]], sem).wait()

    def deg_scatter(c):
        if with_deg:
            pltpu.sync_copy(ones1, dacc.at[dstv2.at[c]], add=True)

    def pair(c0, first, last):
        c1 = c0 + 1
        if not first:
            wait_scatter(c1, rows1, ss1)
        g1 = gather(c1, rows1, sg1)
        wait_gather(c0, rows0, sg0)
        scatter(c0, rows0, ss0)
        deg_scatter(c0)
        g1.wait()
        wait_scatter(c0, rows0, ss0)
        if not last:
            gather(c0 + 2, rows0, sg0)
        scatter(c1, rows1, ss1)
        deg_scatter(c1)

    @pl.loop(0, NSB)
    def _(sb):
        pltpu.sync_copy(e_hbm.at[0, wid, sb], srcv2)
        pltpu.sync_copy(e_hbm.at[1, wid, sb], dstv2)
        gather(0, rows0, sg0)
        pair(0, first=True, last=False)

        @pl.loop(1, SB // 2)
        def _(k):
            pair(2 * k, first=False, last=False)

        wait_scatter(SB - 2, rows1, ss1)
        wait_gather(SB - 1, rows0, sg0)
        pltpu.sync_copy(rows0, acc.at[dstv2.at[SB - 1]], add=True)
        deg_scatter(SB - 1)

    plsc.subcore_barrier()
    pltpu.sync_copy(acc.at[pl.ds(r0, RPS)], part_hbm.at[cid, pl.ds(r0, RPS)])
    if with_deg:
        pltpu.sync_copy(dacc.at[pl.ds(r0, RPS)],
                        deg_hbm.at[pl.ds(cid * NA + r0, RPS)])


def _make_sc_agg(with_deg):
    out_type = [jax.ShapeDtypeStruct((NC, NA, H), jnp.float32)]
    scratch = [
        pltpu.VMEM((SB, CHUNK), jnp.int32),
        pltpu.VMEM((SB, CHUNK), jnp.int32),
        pltpu.VMEM((CHUNK, H), jnp.float32),
        pltpu.VMEM((CHUNK, H), jnp.float32),
    ]
    if with_deg:
        out_type.append(jax.ShapeDtypeStruct((NC * NA,), jnp.float32))
        scratch.append(pltpu.VMEM((CHUNK,), jnp.float32))
    scratch.append(pltpu.VMEM_SHARED((NA, H), jnp.float32))
    if with_deg:
        scratch.append(pltpu.VMEM_SHARED((NA,), jnp.float32))
    scratch += [pltpu.SemaphoreType.DMA] * 4
    return pl.kernel(
        functools.partial(_sc_agg_body, with_deg),
        out_type=out_type,
        mesh=_MESH,
        scratch_types=scratch,
    )


_sc_agg_deg = _make_sc_agg(True)
_sc_agg = _make_sc_agg(False)

BN = 10000


def _mm_body(x_ref, w_ref, o_ref):
    o_ref[...] = jnp.dot(x_ref[...], w_ref[...],
                         preferred_element_type=jnp.float32,
                 precision=jax.lax.Precision.DEFAULT)


def _tc_matmul(x, w):
    n, d = x.shape
    h = w.shape[1]
    return pl.pallas_call(
        _mm_body,
        grid=(n // BN,),
        in_specs=[pl.BlockSpec((BN, d), lambda i: (i, 0)),
                  pl.BlockSpec((d, h), lambda i: (0, 0))],
        out_specs=pl.BlockSpec((BN, h), lambda i: (i, 0)),
        out_shape=jax.ShapeDtypeStruct((n, h), jnp.float32),
    )(x, w)


def _dinv(d_ref):
    deg = d_ref[:, 0:1] + d_ref[:, 1:2]
    return 1.0 / jnp.maximum(deg, 1.0)


def _layer_body(x_ref, ws_ref, wn_ref, b_ref, p_ref, d_ref, h_ref):
    agg = (p_ref[0] + p_ref[1]) * _dinv(d_ref)
    h = jnp.dot(x_ref[...], ws_ref[...], preferred_element_type=jnp.float32,
                 precision=jax.lax.Precision.DEFAULT)
    h = h + jnp.dot(agg, wn_ref[...], preferred_element_type=jnp.float32,
                 precision=jax.lax.Precision.DEFAULT)
    h_ref[...] = jnp.maximum(h + b_ref[...], 0.0)


def _tc_layer(x, w_self, w_neigh, b, part, degT):
    return pl.pallas_call(
        _layer_body,
        grid=(N // BN,),
        in_specs=[pl.BlockSpec((BN, D), lambda i: (i, 0)),
                  pl.BlockSpec((D, H), lambda i: (0, 0)),
                  pl.BlockSpec((D, H), lambda i: (0, 0)),
                  pl.BlockSpec((1, H), lambda i: (0, 0)),
                  pl.BlockSpec((NC, BN, D), lambda i: (0, i, 0)),
                  pl.BlockSpec((BN, NC), lambda i: (i, 0))],
        out_specs=pl.BlockSpec((BN, H), lambda i: (i, 0)),
        out_shape=jax.ShapeDtypeStruct((N, H), jnp.float32),
    )(x, w_self, w_neigh, b, part, degT)


def _final_body(h_ref, ws_ref, wn_ref, b_ref, q_ref, d_ref, wo_ref, bo_ref,
                o_ref):
    agg = (q_ref[0] + q_ref[1]) * _dinv(d_ref)
    h2 = jnp.dot(h_ref[...], ws_ref[...], preferred_element_type=jnp.float32,
                 precision=jax.lax.Precision.DEFAULT)
    h2 = h2 + jnp.dot(agg, wn_ref[...], preferred_element_type=jnp.float32,
                 precision=jax.lax.Precision.DEFAULT)
    h2 = jnp.maximum(h2 + b_ref[...], 0.0)
    o_ref[...] = jnp.dot(h2, wo_ref[...],
                         preferred_element_type=jnp.float32,
                 precision=jax.lax.Precision.DEFAULT) + bo_ref[...]


def _tc_final(h1, w_self2, w_neigh2, b2, part, degT, w_out, b_out):
    return pl.pallas_call(
        _final_body,
        grid=(N // BN,),
        in_specs=[pl.BlockSpec((BN, H), lambda i: (i, 0)),
                  pl.BlockSpec((H, H), lambda i: (0, 0)),
                  pl.BlockSpec((H, H), lambda i: (0, 0)),
                  pl.BlockSpec((1, H), lambda i: (0, 0)),
                  pl.BlockSpec((NC, BN, H), lambda i: (0, i, 0)),
                  pl.BlockSpec((BN, NC), lambda i: (i, 0)),
                  pl.BlockSpec((H, C), lambda i: (0, 0)),
                  pl.BlockSpec((1, C), lambda i: (0, 0))],
        out_specs=pl.BlockSpec((BN, C), lambda i: (i, 0)),
        out_shape=jax.ShapeDtypeStruct((N, C), jnp.float32),
    )(h1, w_self2, w_neigh2, b2, part, degT, w_out, b_out)


def kernel(features, edge_index, W_self1, W_neigh1, b1,
           W_self2, W_neigh2, b2, W_out, b_out):
    e5 = edge_index.reshape(2, NW, NSB, SB, CHUNK)
    z128 = jnp.zeros((RPS, H), jnp.float32)
    z1 = jnp.zeros((RPS,), jnp.float32)

    part1, deg_flat = _sc_agg_deg(features, e5, z128, z1)
    degT = deg_flat.reshape(NC, NA).T
    h1 = _tc_layer(features, W_self1, W_neigh1, b1.reshape(1, H), part1, degT)
    (part2,) = _sc_agg(h1, e5, z128)
    out = _tc_final(h1, W_self2, W_neigh2, b2.reshape(1, H), part2, degT,
                    W_out, b_out.reshape(1, C))
    return out


# speedup vs baseline: 1.0070x; 1.0070x over previous
import functools

import jax
import jax.numpy as jnp
from jax import lax
from jax.experimental import pallas as pl
from jax.experimental.pallas import tpu as pltpu
from jax.experimental.pallas import tpu_sc as plsc

N = 10000
E = 320000
D = 128
H = 128
C = 64

NC = 2
NS = 16
NW = NC * NS
CHUNK = 80
NCHUNK = 125
SB = 25
NSB = NCHUNK // SB
NA = 10240
RPS = NA // NS
NPADR = NA - N

_MESH = plsc.VectorSubcoreMesh(core_axis_name="c", subcore_axis_name="s")


def _sc_agg_body(with_deg, *refs):
    if with_deg:
        (y_hbm, e_hbm, z_hbm, z1_hbm, part_hbm, deg_hbm,
         srcv2, dstv2, rows0, rows1, ones1, acc, dacc,
         sg0, sg1, ss0, ss1) = refs
    else:
        (y_hbm, e_hbm, z_hbm, part_hbm,
         srcv2, dstv2, rows0, rows1, acc,
         sg0, sg1, ss0, ss1) = refs
    cid = lax.axis_index("c")
    sid = lax.axis_index("s")
    wid = cid * NS + sid
    r0 = sid * RPS

    pltpu.sync_copy(z_hbm, acc.at[pl.ds(r0, RPS)])
    if with_deg:
        pltpu.sync_copy(z1_hbm, dacc.at[pl.ds(r0, RPS)])

        @pl.loop(0, CHUNK // 16)
        def _(i):
            ones1[pl.ds(i * 16, 16)] = jnp.full((16,), 1.0, jnp.float32)

    plsc.subcore_barrier()

    def gather(c, rows, sem):
        return pltpu.async_copy(y_hbm.at[srcv2.at[c]], rows, sem)

    def wait_gather(c, rows, sem):
        pltpu.make_async_copy(y_hbm.at[srcv2.at[c]], rows, sem).wait()

    def scatter(c, rows, sem):
        return pltpu.async_copy(rows, acc.at[dstv2.at[c]], sem, add=True)

    def wait_scatter(c, rows, sem):
        pltpu.make_async_copy(rows, acc.at[dstv2.at[c]], sem).wait()

    def deg_scatter(c):
        if with_deg:
            pltpu.sync_copy(ones1, dacc.at[dstv2.at[c]], add=True)

    def pair(c0, first, last):
        c1 = c0 + 1
        if not first:
            wait_scatter(c1, rows1, ss1)
        g1 = gather(c1, rows1, sg1)
        wait_gather(c0, rows0, sg0)
        scatter(c0, rows0, ss0)
        deg_scatter(c0)
        g1.wait()
        wait_scatter(c0, rows0, ss0)
        if not last:
            gather(c0 + 2, rows0, sg0)
        scatter(c1, rows1, ss1)
        deg_scatter(c1)

    @pl.loop(0, NSB)
    def _(sb):
        pltpu.sync_copy(e_hbm.at[0, wid, sb], srcv2)
        pltpu.sync_copy(e_hbm.at[1, wid, sb], dstv2)
        gather(0, rows0, sg0)
        pair(0, first=True, last=False)

        @pl.loop(1, SB // 2)
        def _(k):
            pair(2 * k, first=False, last=False)

        wait_scatter(SB - 2, rows1, ss1)
        wait_gather(SB - 1, rows0, sg0)
        pltpu.sync_copy(rows0, acc.at[dstv2.at[SB - 1]], add=True)
        deg_scatter(SB - 1)

    plsc.subcore_barrier()
    pltpu.sync_copy(acc.at[pl.ds(r0, RPS)], part_hbm.at[cid, pl.ds(r0, RPS)])
    if with_deg:
        pltpu.sync_copy(dacc.at[pl.ds(r0, RPS)],
                        deg_hbm.at[pl.ds(cid * NA + r0, RPS)])


def _make_sc_agg(with_deg):
    out_type = [jax.ShapeDtypeStruct((NC, NA, H), jnp.float32)]
    scratch = [
        pltpu.VMEM((SB, CHUNK), jnp.int32),
        pltpu.VMEM((SB, CHUNK), jnp.int32),
        pltpu.VMEM((CHUNK, H), jnp.float32),
        pltpu.VMEM((CHUNK, H), jnp.float32),
    ]
    if with_deg:
        out_type.append(jax.ShapeDtypeStruct((NC * NA,), jnp.float32))
        scratch.append(pltpu.VMEM((CHUNK,), jnp.float32))
    scratch.append(pltpu.VMEM_SHARED((NA, H), jnp.float32))
    if with_deg:
        scratch.append(pltpu.VMEM_SHARED((NA,), jnp.float32))
    scratch += [pltpu.SemaphoreType.DMA] * 4
    return pl.kernel(
        functools.partial(_sc_agg_body, with_deg),
        out_type=out_type,
        mesh=_MESH,
        scratch_types=scratch,
    )


_sc_agg_deg = _make_sc_agg(True)
_sc_agg = _make_sc_agg(False)

BN = 2000


def _mm_body(x_ref, w_ref, o_ref):
    o_ref[...] = jnp.dot(x_ref[...], w_ref[...],
                         preferred_element_type=jnp.float32,
                 precision=jax.lax.Precision.DEFAULT)


def _tc_matmul(x, w):
    n, d = x.shape
    h = w.shape[1]
    return pl.pallas_call(
        _mm_body,
        grid=(n // BN,),
        in_specs=[pl.BlockSpec((BN, d), lambda i: (i, 0)),
                  pl.BlockSpec((d, h), lambda i: (0, 0))],
        out_specs=pl.BlockSpec((BN, h), lambda i: (i, 0)),
        out_shape=jax.ShapeDtypeStruct((n, h), jnp.float32),
    )(x, w)


def _dinv(d_ref):
    deg = d_ref[:, 0:1] + d_ref[:, 1:2]
    return 1.0 / jnp.maximum(deg, 1.0)


def _layer_body(x_ref, ws_ref, wn_ref, b_ref, p_ref, d_ref, h_ref):
    agg = (p_ref[0] + p_ref[1]) * _dinv(d_ref)
    h = jnp.dot(x_ref[...], ws_ref[...], preferred_element_type=jnp.float32,
                 precision=jax.lax.Precision.DEFAULT)
    h = h + jnp.dot(agg, wn_ref[...], preferred_element_type=jnp.float32,
                 precision=jax.lax.Precision.DEFAULT)
    h_ref[...] = jnp.maximum(h + b_ref[...], 0.0)


def _tc_layer(x, w_self, w_neigh, b, part, degT):
    return pl.pallas_call(
        _layer_body,
        grid=(N // BN,),
        in_specs=[pl.BlockSpec((BN, D), lambda i: (i, 0)),
                  pl.BlockSpec((D, H), lambda i: (0, 0)),
                  pl.BlockSpec((D, H), lambda i: (0, 0)),
                  pl.BlockSpec((1, H), lambda i: (0, 0)),
                  pl.BlockSpec((NC, BN, D), lambda i: (0, i, 0)),
                  pl.BlockSpec((BN, NC), lambda i: (i, 0))],
        out_specs=pl.BlockSpec((BN, H), lambda i: (i, 0)),
        out_shape=jax.ShapeDtypeStruct((N, H), jnp.float32),
    )(x, w_self, w_neigh, b, part, degT)


def _final_body(h_ref, ws_ref, wn_ref, b_ref, q_ref, d_ref, wo_ref, bo_ref,
                o_ref):
    agg = (q_ref[0] + q_ref[1]) * _dinv(d_ref)
    h2 = jnp.dot(h_ref[...], ws_ref[...], preferred_element_type=jnp.float32,
                 precision=jax.lax.Precision.DEFAULT)
    h2 = h2 + jnp.dot(agg, wn_ref[...], preferred_element_type=jnp.float32,
                 precision=jax.lax.Precision.DEFAULT)
    h2 = jnp.maximum(h2 + b_ref[...], 0.0)
    o_ref[...] = jnp.dot(h2, wo_ref[...],
                         preferred_element_type=jnp.float32,
                 precision=jax.lax.Precision.DEFAULT) + bo_ref[...]


def _tc_final(h1, w_self2, w_neigh2, b2, part, degT, w_out, b_out):
    return pl.pallas_call(
        _final_body,
        grid=(N // BN,),
        in_specs=[pl.BlockSpec((BN, H), lambda i: (i, 0)),
                  pl.BlockSpec((H, H), lambda i: (0, 0)),
                  pl.BlockSpec((H, H), lambda i: (0, 0)),
                  pl.BlockSpec((1, H), lambda i: (0, 0)),
                  pl.BlockSpec((NC, BN, H), lambda i: (0, i, 0)),
                  pl.BlockSpec((BN, NC), lambda i: (i, 0)),
                  pl.BlockSpec((H, C), lambda i: (0, 0)),
                  pl.BlockSpec((1, C), lambda i: (0, 0))],
        out_specs=pl.BlockSpec((BN, C), lambda i: (i, 0)),
        out_shape=jax.ShapeDtypeStruct((N, C), jnp.float32),
    )(h1, w_self2, w_neigh2, b2, part, degT, w_out, b_out)


def kernel(features, edge_index, W_self1, W_neigh1, b1,
           W_self2, W_neigh2, b2, W_out, b_out):
    e5 = edge_index.reshape(2, NW, NSB, SB, CHUNK)
    z128 = jnp.zeros((RPS, H), jnp.float32)
    z1 = jnp.zeros((RPS,), jnp.float32)

    part1, deg_flat = _sc_agg_deg(features, e5, z128, z1)
    degT = deg_flat.reshape(NC, NA).T
    h1 = _tc_layer(features, W_self1, W_neigh1, b1.reshape(1, H), part1, degT)
    (part2,) = _sc_agg(h1, e5, z128)
    out = _tc_final(h1, W_self2, W_neigh2, b2.reshape(1, H), part2, degT,
                    W_out, b_out.reshape(1, C))
    return out
